# baseline (speedup 1.0000x reference)
import functools

import jax
import jax.numpy as jnp
from jax import lax
from jax.experimental import pallas as pl
from jax.experimental.pallas import tpu as pltpu
from jax.experimental.pallas import tpu_sc as plsc

NC, NS, L = 2, 16, 16
NW = NC * NS


def _sc_body(T, B, D, CHUNK,
             tef_hbm, t_hbm, dt_hbm, y_hbm, yn_hbm, idx_hbm, out_hbm,
             idx_v, t_v, dt_v, th_v, tef_v, yn_v, zbuf_v, vals_v,
             zsem, ssem):
    wid = lax.axis_index("s") * NC + lax.axis_index("c")
    base = wid * CHUNK

    zrow = jnp.zeros((L,), jnp.float32)

    def zbody(i, _):
        for k in range(D // L):
            zbuf_v[pl.ds(i * D + k * L, L)] = zrow
        return 0
    lax.fori_loop(0, CHUNK, zbody, 0)

    zcopies = [
        pltpu.make_async_copy(
            zbuf_v, out_hbm.at[pl.ds((t * B + base) * D, CHUNK * D)], zsem)
        for t in range(T)
    ]
    for c in zcopies:
        c.start()

    pltpu.sync_copy(idx_hbm.at[pl.ds(base, CHUNK)], idx_v)
    pltpu.sync_copy(t_hbm.at[pl.ds(base, CHUNK)], t_v)
    pltpu.sync_copy(dt_hbm.at[pl.ds(base, CHUNK)], dt_v)
    pltpu.sync_copy(tef_hbm.at[pl.ds(base * T, CHUNK * T)], tef_v)
    pltpu.sync_copy(yn_hbm.at[pl.ds(base * D, CHUNK * D)], yn_v)
    pltpu.sync_copy(y_hbm.at[pl.ds(base * D, CHUNK * D)], vals_v)

    iota = lax.broadcasted_iota(jnp.int32, (L,), 0)
    for j in range(CHUNK // L):
        sl = pl.ds(j * L, L)
        fi16 = (iota + j * L) * T + idx_v[sl]
        te16 = plsc.load_gather(tef_v, [fi16])
        th = (te16 - t_v[sl]) / dt_v[sl]
        th_v[sl] = jnp.minimum(jnp.maximum(th, 0.0), 1.0)

    def fbody(i, _):
        th16 = plsc.load_gather(th_v, [jnp.zeros((L,), jnp.int32) + i])
        for k in range(D // L):
            sl = pl.ds(i * D + k * L, L)
            yv = vals_v[sl]
            ynv = yn_v[sl]
            vals_v[sl] = yv + th16 * (ynv - yv)
        return 0
    lax.fori_loop(0, CHUNK, fbody, 0)

    for c in zcopies:
        c.wait()

    def sbody(j, _):
        idx16 = idx_v[pl.ds(j * L, L)]
        for l in range(L):
            i = j * L + l
            r = (idx16[l] * B + base + i) * D
            pltpu.make_async_copy(
                vals_v.at[pl.ds(i * D, D)],
                out_hbm.at[pl.ds(r, D)], ssem).start()
        return 0
    lax.fori_loop(0, CHUNK // L, sbody, 0)

    pltpu.make_async_copy(
        y_hbm.at[pl.ds(0, CHUNK * D)], vals_v, ssem).wait()


def kernel(y_eval, t_eval, t, dt, y, y_next, eval_t_idx, sample_idx):
    T, B, D = y_eval.shape
    CHUNK = B // NW
    mesh = plsc.VectorSubcoreMesh(
        core_axis_name="c", subcore_axis_name="s",
        num_cores=NC, num_subcores=NS)

    k = functools.partial(
        pl.kernel,
        out_type=jax.ShapeDtypeStruct((T * B * D,), jnp.float32),
        mesh=mesh,
        scratch_types=[
            pltpu.VMEM((CHUNK,), jnp.int32),
            pltpu.VMEM((CHUNK,), jnp.float32),
            pltpu.VMEM((CHUNK,), jnp.float32),
            pltpu.VMEM((CHUNK,), jnp.float32),
            pltpu.VMEM((CHUNK * T,), jnp.float32),
            pltpu.VMEM((CHUNK * D,), jnp.float32),
            pltpu.VMEM((CHUNK * D,), jnp.float32),
            pltpu.VMEM((CHUNK * D,), jnp.float32),
            pltpu.SemaphoreType.DMA,
            pltpu.SemaphoreType.DMA,
        ],
        compiler_params=pltpu.CompilerParams(needs_layout_passes=False),
    )(functools.partial(_sc_body, T, B, D, CHUNK))

    out = k(t_eval.reshape(B * T), t, dt, y.reshape(B * D),
            y_next.reshape(B * D), eval_t_idx)
    return out.reshape(T, B, D)
